# baseline (speedup 1.0000x reference)
import functools

import jax
import jax.numpy as jnp
from jax.experimental import pallas as pl


def _sl1_sum(x):
    ax = jnp.abs(x)
    return jnp.sum(jnp.where(ax < 1.0, 0.5 * x * x, ax - 0.5),
                   axis=(0, 1), keepdims=True)


def _sl1_elt(x):
    ax = jnp.abs(x)
    return jnp.where(ax < 1.0, 0.5 * x * x, ax - 0.5)


def _onehot_dot(onehot_bf, coords_bf):
    return jax.lax.dot_general(onehot_bf, coords_bf, (((1,), (0,)), ((), ())),
                               preferred_element_type=jnp.float32)


def _f32(x):
    return jax.lax.bitcast_convert_type(x, jnp.float32)


def _i32(x):
    return jax.lax.bitcast_convert_type(x, jnp.int32)


def _one_pair(predT, tagT, w, consts, *, N, M, C, B, I):
    lane_c, srow_c, srow, mask = consts

    tag = jnp.transpose(tagT, (1, 0))
    predT_bf = predT.astype(jnp.bfloat16)
    tagT_bf = tagT.astype(jnp.bfloat16)
    p_row = [predT[d : d + 1, :] for d in range(3)]

    run_key = jnp.full((1, N), jnp.inf, jnp.float32)
    run_tile = jnp.zeros((1, N), jnp.int32)
    tmp2_sum = jnp.zeros((1, 1), jnp.float32)

    num_tiles = M // C
    for jb in range(num_tiles):
        c0 = jb * C
        tag_blk = tag[c0 : c0 + C, :]
        t_col = [tag_blk[:, d : d + 1] for d in range(3)]

        d0 = t_col[0] - p_row[0]
        d2m = d0 * d0
        d1 = t_col[1] - p_row[1]
        d2m = d2m + d1 * d1
        dd = t_col[2] - p_row[2]
        d2m = d2m + dd * dd

        kb = _i32(d2m) & mask

        key_c = _f32(kb + lane_c)
        kmin_c = jnp.min(key_c, axis=1, keepdims=True)
        csel = jnp.where(key_c == kmin_c, 1.0, 0.0).astype(jnp.bfloat16)
        pp = jax.lax.dot_general(csel, predT_bf, (((1,), (1,)), ((), ())),
                                 preferred_element_type=jnp.float32)
        tmp2_sum = tmp2_sum + _sl1_sum(tag_blk - pp)

        key_r = _f32(kb + srow_c)
        kmin_r = jnp.min(key_r, axis=0, keepdims=True)
        upd = kmin_r < run_key
        run_key = jnp.where(upd, kmin_r, run_key)
        run_tile = jnp.where(upd, jb, run_tile)

    run_local = (_i32(run_key) & jnp.int32(C - 1)) + run_tile * C
    bt = jnp.zeros((3, N), jnp.float32)
    for jb in range(num_tiles):
        c0 = jb * C
        oh = jnp.where(srow == run_local - c0, 1.0, 0.0).astype(jnp.bfloat16)
        bt = bt + _onehot_dot(tagT_bf[:, c0 : c0 + C], oh)

    tmp1_sum = _sl1_sum(predT - bt)

    cp = jnp.sum(predT, axis=1, keepdims=True) / N
    ct = jnp.sum(tagT, axis=1, keepdims=True) / M
    csum = jnp.sum(_sl1_elt(cp - ct), axis=(0, 1), keepdims=True)

    pair = w * (tmp1_sum / (3.0 * N) + tmp2_sum / (3.0 * M))
    return pair / B, csum / (B * 3.0)


def _step_body(predT_ref, tagT_ref, w_ref,
               loss_ref, lossc_ref, *, N, M, C, B, I, P):
    g = pl.program_id(0)

    @pl.when(g == 0)
    def _init():
        loss_ref[...] = jnp.zeros((1, 1), jnp.float32)
        lossc_ref[...] = jnp.zeros((1, 1), jnp.float32)

    bias = jnp.int32(0x800000)
    lane_c = jax.lax.broadcasted_iota(jnp.int32, (C, N), 1) + bias
    srow_c = jax.lax.broadcasted_iota(jnp.int32, (C, N), 0) + bias
    srow = jax.lax.broadcasted_iota(jnp.int32, (C, N), 0)
    mask = jnp.int32(-1024)
    consts = (lane_c, srow_c, srow, mask)

    acc = jnp.zeros((1, 1), jnp.float32)
    accc = jnp.zeros((1, 1), jnp.float32)
    for p in range(P):
        dl, dc = _one_pair(predT_ref[p], tagT_ref[p], w_ref[p], consts,
                           N=N, M=M, C=C, B=B, I=I)
        acc = acc + dl
        accc = accc + dc
    loss_ref[...] += acc
    lossc_ref[...] += accc


def kernel(X_v, target_X_v, weights, device=0):
    B, I, N, D = X_v.shape
    M = target_X_v.shape[2]
    G = B * I

    predT = jnp.transpose(X_v.reshape(G, N, D), (0, 2, 1))
    tagT = jnp.transpose(target_X_v.reshape(G, M, D), (0, 2, 1))
    w = weights.reshape(G, 1, 1).astype(jnp.float32)

    C = 128
    P = 8

    body = functools.partial(_step_body, N=N, M=M, C=C, B=B, I=I, P=P)
    loss, lossc = pl.pallas_call(
        body,
        grid=(G // P,),
        in_specs=[
            pl.BlockSpec((P, D, N), lambda g: (g, 0, 0)),
            pl.BlockSpec((P, D, M), lambda g: (g, 0, 0)),
            pl.BlockSpec((P, 1, 1), lambda g: (g, 0, 0)),
        ],
        out_specs=[
            pl.BlockSpec((1, 1), lambda g: (0, 0)),
            pl.BlockSpec((1, 1), lambda g: (0, 0)),
        ],
        out_shape=[
            jax.ShapeDtypeStruct((1, 1), jnp.float32),
            jax.ShapeDtypeStruct((1, 1), jnp.float32),
        ],
    )(predT, tagT, w)

    return (loss[0, 0], lossc[0, 0])


# speedup vs baseline: 1.1154x; 1.1154x over previous
import functools

import jax
import jax.numpy as jnp
from jax.experimental import pallas as pl


def _sl1_sum(x):
    ax = jnp.abs(x)
    return jnp.sum(jnp.where(ax < 1.0, 0.5 * x * x, ax - 0.5),
                   axis=(0, 1), keepdims=True)


def _sl1_elt(x):
    ax = jnp.abs(x)
    return jnp.where(ax < 1.0, 0.5 * x * x, ax - 0.5)


def _onehot_dot(onehot_bf, coords_bf):
    return jax.lax.dot_general(onehot_bf, coords_bf, (((1,), (0,)), ((), ())),
                               preferred_element_type=jnp.float32)


def _f32(x):
    return jax.lax.bitcast_convert_type(x, jnp.float32)


def _i32(x):
    return jax.lax.bitcast_convert_type(x, jnp.int32)


def _one_pair(predT, pred, tag, tagT, w, consts, *, N, M, C, B, I):
    lane_c, srow_c, srow, mask = consts

    pred_bf = pred.astype(jnp.bfloat16)
    tagT_bf = tagT.astype(jnp.bfloat16)
    p_row = [predT[d : d + 1, :] for d in range(3)]

    run_key = jnp.full((1, N), jnp.inf, jnp.float32)
    run_tile = jnp.zeros((1, N), jnp.int32)
    tmp2_sum = jnp.zeros((1, 1), jnp.float32)

    num_tiles = M // C
    for jb in range(num_tiles):
        c0 = jb * C
        tag_blk = tag[c0 : c0 + C, :]
        t_col = [tag_blk[:, d : d + 1] for d in range(3)]

        d0 = t_col[0] - p_row[0]
        d2m = d0 * d0
        d1 = t_col[1] - p_row[1]
        d2m = d2m + d1 * d1
        dd = t_col[2] - p_row[2]
        d2m = d2m + dd * dd

        kb = _i32(d2m) & mask

        key_c = _f32(kb + lane_c)
        kmin_c = jnp.min(key_c, axis=1, keepdims=True)
        csel = jnp.where(key_c == kmin_c, 1.0, 0.0).astype(jnp.bfloat16)
        pp = _onehot_dot(csel, pred_bf)
        tmp2_sum = tmp2_sum + _sl1_sum(tag_blk - pp)

        key_r = _f32(kb + srow_c)
        kmin_r = jnp.min(key_r, axis=0, keepdims=True)
        upd = kmin_r < run_key
        run_key = jnp.where(upd, kmin_r, run_key)
        run_tile = jnp.where(upd, jb, run_tile)

    run_local = (_i32(run_key) & jnp.int32(C - 1)) + run_tile * C
    bt = jnp.zeros((3, N), jnp.float32)
    for jb in range(num_tiles):
        c0 = jb * C
        oh = jnp.where(srow == run_local - c0, 1.0, 0.0).astype(jnp.bfloat16)
        bt = bt + _onehot_dot(tagT_bf[:, c0 : c0 + C], oh)

    tmp1_sum = _sl1_sum(predT - bt)

    cp = jnp.sum(predT, axis=1, keepdims=True) / N
    ct = jnp.sum(tagT, axis=1, keepdims=True) / M
    csum = jnp.sum(_sl1_elt(cp - ct), axis=(0, 1), keepdims=True)

    pair = w * (tmp1_sum / (3.0 * N) + tmp2_sum / (3.0 * M))
    return pair / B, csum / (B * 3.0)


def _step_body(predT_ref, pred_ref, tag_ref, tagT_ref, w_ref,
               loss_ref, lossc_ref, *, N, M, C, B, I, P):
    g = pl.program_id(0)

    @pl.when(g == 0)
    def _init():
        loss_ref[...] = jnp.zeros((1, 1), jnp.float32)
        lossc_ref[...] = jnp.zeros((1, 1), jnp.float32)

    bias = jnp.int32(0x800000)
    lane_c = jax.lax.broadcasted_iota(jnp.int32, (C, N), 1) + bias
    srow_c = jax.lax.broadcasted_iota(jnp.int32, (C, N), 0) + bias
    srow = jax.lax.broadcasted_iota(jnp.int32, (C, N), 0)
    mask = jnp.int32(-1024)
    consts = (lane_c, srow_c, srow, mask)

    acc = jnp.zeros((1, 1), jnp.float32)
    accc = jnp.zeros((1, 1), jnp.float32)
    for p in range(P):
        dl, dc = _one_pair(predT_ref[p], pred_ref[p], tag_ref[p],
                           tagT_ref[p], w_ref[p], consts,
                           N=N, M=M, C=C, B=B, I=I)
        acc = acc + dl
        accc = accc + dc
    loss_ref[...] += acc
    lossc_ref[...] += accc


def kernel(X_v, target_X_v, weights, device=0):
    B, I, N, D = X_v.shape
    M = target_X_v.shape[2]
    G = B * I

    pred = X_v.reshape(G, N, D)
    predT = jnp.transpose(pred, (0, 2, 1))
    tag = target_X_v.reshape(G, M, D)
    tagT = jnp.transpose(tag, (0, 2, 1))
    w = weights.reshape(G, 1, 1).astype(jnp.float32)

    C = 128
    P = 8

    body = functools.partial(_step_body, N=N, M=M, C=C, B=B, I=I, P=P)
    loss, lossc = pl.pallas_call(
        body,
        grid=(G // P,),
        in_specs=[
            pl.BlockSpec((P, D, N), lambda g: (g, 0, 0)),
            pl.BlockSpec((P, N, D), lambda g: (g, 0, 0)),
            pl.BlockSpec((P, M, D), lambda g: (g, 0, 0)),
            pl.BlockSpec((P, D, M), lambda g: (g, 0, 0)),
            pl.BlockSpec((P, 1, 1), lambda g: (g, 0, 0)),
        ],
        out_specs=[
            pl.BlockSpec((1, 1), lambda g: (0, 0)),
            pl.BlockSpec((1, 1), lambda g: (0, 0)),
        ],
        out_shape=[
            jax.ShapeDtypeStruct((1, 1), jnp.float32),
            jax.ShapeDtypeStruct((1, 1), jnp.float32),
        ],
    )(predT, pred, tag, tagT, w)

    return (loss[0, 0], lossc[0, 0])


# speedup vs baseline: 1.1882x; 1.0653x over previous
import functools

import jax
import jax.numpy as jnp
from jax.experimental import pallas as pl


def _sl1_sum(x):
    ax = jnp.abs(x)
    return jnp.sum(jnp.where(ax < 1.0, 0.5 * x * x, ax - 0.5),
                   axis=(0, 1), keepdims=True)


def _sl1_elt(x):
    ax = jnp.abs(x)
    return jnp.where(ax < 1.0, 0.5 * x * x, ax - 0.5)


def _onehot_dot(onehot_bf, coords_bf):
    return jax.lax.dot_general(onehot_bf, coords_bf, (((1,), (0,)), ((), ())),
                               preferred_element_type=jnp.float32)


def _f32(x):
    return jax.lax.bitcast_convert_type(x, jnp.float32)


def _i32(x):
    return jax.lax.bitcast_convert_type(x, jnp.int32)


def _one_pair(predT, pred, tag, tagT, w, consts, *, N, M, C, B, I):
    lane_c, srow_c, srow, mask = consts

    pred_bf = pred.astype(jnp.bfloat16)
    tagT_bf = tagT.astype(jnp.bfloat16)
    p_row = [predT[d : d + 1, :] for d in range(3)]

    run_key = jnp.full((1, N), jnp.inf, jnp.float32)
    run_tile = jnp.zeros((1, N), jnp.int32)
    tmp2_sum = jnp.zeros((1, 1), jnp.float32)

    num_tiles = M // C
    for jb in range(num_tiles):
        c0 = jb * C
        tag_blk = tag[c0 : c0 + C, :]
        t_col = [tag_blk[:, d : d + 1] for d in range(3)]

        d0 = t_col[0] - p_row[0]
        d2m = d0 * d0
        d1 = t_col[1] - p_row[1]
        d2m = d2m + d1 * d1
        dd = t_col[2] - p_row[2]
        d2m = d2m + dd * dd

        kb = _i32(d2m) & mask

        key_c = _f32(kb + lane_c)
        kmin_c = jnp.min(key_c, axis=1, keepdims=True)
        csel = jnp.where(key_c == kmin_c, 1.0, 0.0).astype(jnp.bfloat16)
        pp = _onehot_dot(csel, pred_bf)
        tmp2_sum = tmp2_sum + _sl1_sum(tag_blk - pp)

        key_r = _f32(kb + srow_c)
        kmin_r = jnp.min(key_r, axis=0, keepdims=True)
        upd = kmin_r < run_key
        run_key = jnp.where(upd, kmin_r, run_key)
        run_tile = jnp.where(upd, jb, run_tile)

    run_local = (_i32(run_key) & jnp.int32(C - 1)) + run_tile * C
    bt = jnp.zeros((3, N), jnp.float32)
    for jb in range(num_tiles):
        c0 = jb * C
        oh = jnp.where(srow == run_local - c0, 1.0, 0.0).astype(jnp.bfloat16)
        bt = bt + _onehot_dot(tagT_bf[:, c0 : c0 + C], oh)

    tmp1_sum = _sl1_sum(predT - bt)

    cp = jnp.sum(predT, axis=1, keepdims=True) / N
    ct = jnp.sum(tagT, axis=1, keepdims=True) / M
    csum = jnp.sum(_sl1_elt(cp - ct), axis=(0, 1), keepdims=True)

    pair = w * (tmp1_sum / (3.0 * N) + tmp2_sum / (3.0 * M))
    return pair / B, csum / (B * 3.0)


def _step_body(predT_ref, pred_ref, tag_ref, tagT_ref, w_ref,
               loss_ref, lossc_ref, *, N, M, C, B, I, P):
    g = pl.program_id(0)

    @pl.when(g == 0)
    def _init():
        loss_ref[...] = jnp.zeros((1, 1), jnp.float32)
        lossc_ref[...] = jnp.zeros((1, 1), jnp.float32)

    bias = jnp.int32(0x800000)
    lane_c = jax.lax.broadcasted_iota(jnp.int32, (C, N), 1) + bias
    srow_c = jax.lax.broadcasted_iota(jnp.int32, (C, N), 0) + bias
    srow = jax.lax.broadcasted_iota(jnp.int32, (C, N), 0)
    mask = jnp.int32(-1024)
    consts = (lane_c, srow_c, srow, mask)

    acc = jnp.zeros((1, 1), jnp.float32)
    accc = jnp.zeros((1, 1), jnp.float32)
    for p in range(P):
        dl, dc = _one_pair(predT_ref[p], pred_ref[p], tag_ref[p],
                           tagT_ref[p], w_ref[p], consts,
                           N=N, M=M, C=C, B=B, I=I)
        acc = acc + dl
        accc = accc + dc
    loss_ref[...] += acc
    lossc_ref[...] += accc


def kernel(X_v, target_X_v, weights, device=0):
    B, I, N, D = X_v.shape
    M = target_X_v.shape[2]
    G = B * I

    pred = X_v.reshape(G, N, D)
    predT = jnp.transpose(pred, (0, 2, 1))
    tag = target_X_v.reshape(G, M, D)
    tagT = jnp.transpose(tag, (0, 2, 1))
    w = weights.reshape(G, 1, 1).astype(jnp.float32)

    C = 256
    P = 8

    body = functools.partial(_step_body, N=N, M=M, C=C, B=B, I=I, P=P)
    loss, lossc = pl.pallas_call(
        body,
        grid=(G // P,),
        in_specs=[
            pl.BlockSpec((P, D, N), lambda g: (g, 0, 0)),
            pl.BlockSpec((P, N, D), lambda g: (g, 0, 0)),
            pl.BlockSpec((P, M, D), lambda g: (g, 0, 0)),
            pl.BlockSpec((P, D, M), lambda g: (g, 0, 0)),
            pl.BlockSpec((P, 1, 1), lambda g: (g, 0, 0)),
        ],
        out_specs=[
            pl.BlockSpec((1, 1), lambda g: (0, 0)),
            pl.BlockSpec((1, 1), lambda g: (0, 0)),
        ],
        out_shape=[
            jax.ShapeDtypeStruct((1, 1), jnp.float32),
            jax.ShapeDtypeStruct((1, 1), jnp.float32),
        ],
    )(predT, pred, tag, tagT, w)

    return (loss[0, 0], lossc[0, 0])


# speedup vs baseline: 1.2541x; 1.0555x over previous
import functools

import jax
import jax.numpy as jnp
from jax.experimental import pallas as pl


def _sl1_sum(x):
    ax = jnp.abs(x)
    return jnp.sum(jnp.where(ax < 1.0, 0.5 * x * x, ax - 0.5),
                   axis=(0, 1), keepdims=True)


def _sl1_elt(x):
    ax = jnp.abs(x)
    return jnp.where(ax < 1.0, 0.5 * x * x, ax - 0.5)


def _onehot_dot(onehot_bf, coords_bf):
    return jax.lax.dot_general(onehot_bf, coords_bf, (((1,), (0,)), ((), ())),
                               preferred_element_type=jnp.float32)


def _f32(x):
    return jax.lax.bitcast_convert_type(x, jnp.float32)


def _i32(x):
    return jax.lax.bitcast_convert_type(x, jnp.int32)


def _one_pair(predT, pred, tag, tagT, w, consts, *, N, M, C, B, I):
    lane_c, srow_c, srow, mask = consts

    pred_bf = pred.astype(jnp.bfloat16)
    tagT_bf = tagT.astype(jnp.bfloat16)
    p_row = [predT[d : d + 1, :] for d in range(3)]

    run_key = jnp.full((1, N), jnp.inf, jnp.float32)
    run_tile = jnp.zeros((1, N), jnp.int32)
    tmp2_sum = jnp.zeros((1, 1), jnp.float32)

    num_tiles = M // C
    for jb in range(num_tiles):
        c0 = jb * C
        tag_blk = tag[c0 : c0 + C, :]
        t_col = [tag_blk[:, d : d + 1] for d in range(3)]

        d0 = t_col[0] - p_row[0]
        d2m = d0 * d0
        d1 = t_col[1] - p_row[1]
        d2m = d2m + d1 * d1
        dd = t_col[2] - p_row[2]
        d2m = d2m + dd * dd

        kb = _i32(d2m) & mask

        key_c = _f32(kb + lane_c)
        kmin_c = jnp.min(key_c, axis=1, keepdims=True)
        csel = jnp.where(key_c == kmin_c, 1.0, 0.0).astype(jnp.bfloat16)
        pp = _onehot_dot(csel, pred_bf)
        tmp2_sum = tmp2_sum + _sl1_sum(tag_blk - pp)

        key_r = _f32(kb + srow_c)
        kmin_r = jnp.min(key_r, axis=0, keepdims=True)
        upd = kmin_r < run_key
        run_key = jnp.where(upd, kmin_r, run_key)
        run_tile = jnp.where(upd, jb, run_tile)

    run_local = (_i32(run_key) & jnp.int32(C - 1)) + run_tile * C
    bt = jnp.zeros((3, N), jnp.float32)
    for jb in range(num_tiles):
        c0 = jb * C
        oh = jnp.where(srow == run_local - c0, 1.0, 0.0).astype(jnp.bfloat16)
        bt = bt + _onehot_dot(tagT_bf[:, c0 : c0 + C], oh)

    tmp1_sum = _sl1_sum(predT - bt)

    cp = jnp.sum(predT, axis=1, keepdims=True) / N
    ct = jnp.sum(tagT, axis=1, keepdims=True) / M
    csum = jnp.sum(_sl1_elt(cp - ct), axis=(0, 1), keepdims=True)

    pair = w * (tmp1_sum / (3.0 * N) + tmp2_sum / (3.0 * M))
    return pair / B, csum / (B * 3.0)


def _step_body(predT_ref, pred_ref, tag_ref, tagT_ref, w_ref,
               loss_ref, lossc_ref, *, N, M, C, B, I, P):
    g = pl.program_id(0)

    @pl.when(g == 0)
    def _init():
        loss_ref[...] = jnp.zeros((1, 1), jnp.float32)
        lossc_ref[...] = jnp.zeros((1, 1), jnp.float32)

    bias = jnp.int32(0x800000)
    lane_c = jax.lax.broadcasted_iota(jnp.int32, (C, N), 1) + bias
    srow_c = jax.lax.broadcasted_iota(jnp.int32, (C, N), 0) + bias
    srow = jax.lax.broadcasted_iota(jnp.int32, (C, N), 0)
    mask = jnp.int32(-1024)
    consts = (lane_c, srow_c, srow, mask)

    acc = jnp.zeros((1, 1), jnp.float32)
    accc = jnp.zeros((1, 1), jnp.float32)
    for p in range(P):
        dl, dc = _one_pair(predT_ref[p], pred_ref[p], tag_ref[p],
                           tagT_ref[p], w_ref[p], consts,
                           N=N, M=M, C=C, B=B, I=I)
        acc = acc + dl
        accc = accc + dc
    loss_ref[...] += acc
    lossc_ref[...] += accc


def kernel(X_v, target_X_v, weights, device=0):
    B, I, N, D = X_v.shape
    M = target_X_v.shape[2]
    G = B * I

    pred = X_v.reshape(G, N, D)
    predT = jnp.transpose(pred, (0, 2, 1))
    tag = target_X_v.reshape(G, M, D)
    tagT = jnp.transpose(tag, (0, 2, 1))
    w = weights.reshape(G, 1, 1).astype(jnp.float32)

    C = 512
    P = 8

    body = functools.partial(_step_body, N=N, M=M, C=C, B=B, I=I, P=P)
    loss, lossc = pl.pallas_call(
        body,
        grid=(G // P,),
        in_specs=[
            pl.BlockSpec((P, D, N), lambda g: (g, 0, 0)),
            pl.BlockSpec((P, N, D), lambda g: (g, 0, 0)),
            pl.BlockSpec((P, M, D), lambda g: (g, 0, 0)),
            pl.BlockSpec((P, D, M), lambda g: (g, 0, 0)),
            pl.BlockSpec((P, 1, 1), lambda g: (g, 0, 0)),
        ],
        out_specs=[
            pl.BlockSpec((1, 1), lambda g: (0, 0)),
            pl.BlockSpec((1, 1), lambda g: (0, 0)),
        ],
        out_shape=[
            jax.ShapeDtypeStruct((1, 1), jnp.float32),
            jax.ShapeDtypeStruct((1, 1), jnp.float32),
        ],
    )(predT, pred, tag, tagT, w)

    return (loss[0, 0], lossc[0, 0])


# speedup vs baseline: 1.2732x; 1.0153x over previous
import functools

import jax
import jax.numpy as jnp
from jax.experimental import pallas as pl


def _sl1_sum(x):
    ax = jnp.abs(x)
    return jnp.sum(jnp.where(ax < 1.0, 0.5 * x * x, ax - 0.5),
                   axis=(0, 1), keepdims=True)


def _sl1_elt(x):
    ax = jnp.abs(x)
    return jnp.where(ax < 1.0, 0.5 * x * x, ax - 0.5)


def _onehot_dot(onehot_bf, coords_bf):
    return jax.lax.dot_general(onehot_bf, coords_bf, (((1,), (0,)), ((), ())),
                               preferred_element_type=jnp.float32)


def _f32(x):
    return jax.lax.bitcast_convert_type(x, jnp.float32)


def _i32(x):
    return jax.lax.bitcast_convert_type(x, jnp.int32)


def _one_pair(predT, pred, tag, tagT, w, consts, *, N, M, C, B, I):
    lane_c, srow_c, srow, mask = consts

    pred_bf = pred.astype(jnp.bfloat16)
    tagT_bf = tagT.astype(jnp.bfloat16)
    p_row = [predT[d : d + 1, :] for d in range(3)]

    run_key = jnp.full((1, N), jnp.inf, jnp.float32)
    run_tile = jnp.zeros((1, N), jnp.int32)
    tmp2_sum = jnp.zeros((1, 1), jnp.float32)

    num_tiles = M // C
    for jb in range(num_tiles):
        c0 = jb * C
        tag_blk = tag[c0 : c0 + C, :]
        t_col = [tag_blk[:, d : d + 1] for d in range(3)]

        d0 = t_col[0] - p_row[0]
        d2m = d0 * d0
        d1 = t_col[1] - p_row[1]
        d2m = d2m + d1 * d1
        dd = t_col[2] - p_row[2]
        d2m = d2m + dd * dd

        kb = _i32(d2m) & mask

        key_c = _f32(kb + lane_c)
        kmin_c = jnp.min(key_c, axis=1, keepdims=True)
        csel = jnp.where(key_c == kmin_c, 1.0, 0.0).astype(jnp.bfloat16)
        pp = _onehot_dot(csel, pred_bf)
        tmp2_sum = tmp2_sum + _sl1_sum(tag_blk - pp)

        key_r = _f32(kb + srow_c)
        kmin_r = jnp.min(key_r, axis=0, keepdims=True)
        upd = kmin_r < run_key
        run_key = jnp.where(upd, kmin_r, run_key)
        run_tile = jnp.where(upd, jb, run_tile)

    run_local = (_i32(run_key) & jnp.int32(C - 1)) + run_tile * C
    bt = jnp.zeros((3, N), jnp.float32)
    for jb in range(num_tiles):
        c0 = jb * C
        oh = jnp.where(srow == run_local - c0, 1.0, 0.0).astype(jnp.bfloat16)
        bt = bt + _onehot_dot(tagT_bf[:, c0 : c0 + C], oh)

    tmp1_sum = _sl1_sum(predT - bt)

    cp = jnp.sum(predT, axis=1, keepdims=True) / N
    ct = jnp.sum(tagT, axis=1, keepdims=True) / M
    csum = jnp.sum(_sl1_elt(cp - ct), axis=(0, 1), keepdims=True)

    pair = w * (tmp1_sum / (3.0 * N) + tmp2_sum / (3.0 * M))
    return pair / B, csum / (B * 3.0)


def _step_body(predT_ref, pred_ref, tag_ref, tagT_ref, w_ref,
               loss_ref, lossc_ref, *, N, M, C, B, I, P):
    g = pl.program_id(0)

    @pl.when(g == 0)
    def _init():
        loss_ref[...] = jnp.zeros((1, 1), jnp.float32)
        lossc_ref[...] = jnp.zeros((1, 1), jnp.float32)

    bias = jnp.int32(0x800000)
    lane_c = jax.lax.broadcasted_iota(jnp.int32, (C, N), 1) + bias
    srow_c = jax.lax.broadcasted_iota(jnp.int32, (C, N), 0) + bias
    srow = jax.lax.broadcasted_iota(jnp.int32, (C, N), 0)
    mask = jnp.int32(-1024)
    consts = (lane_c, srow_c, srow, mask)

    acc = jnp.zeros((1, 1), jnp.float32)
    accc = jnp.zeros((1, 1), jnp.float32)
    for p in range(P):
        dl, dc = _one_pair(predT_ref[p], pred_ref[p], tag_ref[p],
                           tagT_ref[p], w_ref[p], consts,
                           N=N, M=M, C=C, B=B, I=I)
        acc = acc + dl
        accc = accc + dc
    loss_ref[...] += acc
    lossc_ref[...] += accc


def kernel(X_v, target_X_v, weights, device=0):
    B, I, N, D = X_v.shape
    M = target_X_v.shape[2]
    G = B * I

    pred = X_v.reshape(G, N, D)
    predT = jnp.transpose(pred, (0, 2, 1))
    tag = target_X_v.reshape(G, M, D)
    tagT = jnp.transpose(tag, (0, 2, 1))
    w = weights.reshape(G, 1, 1).astype(jnp.float32)

    C = 1024
    P = 8

    body = functools.partial(_step_body, N=N, M=M, C=C, B=B, I=I, P=P)
    loss, lossc = pl.pallas_call(
        body,
        grid=(G // P,),
        in_specs=[
            pl.BlockSpec((P, D, N), lambda g: (g, 0, 0)),
            pl.BlockSpec((P, N, D), lambda g: (g, 0, 0)),
            pl.BlockSpec((P, M, D), lambda g: (g, 0, 0)),
            pl.BlockSpec((P, D, M), lambda g: (g, 0, 0)),
            pl.BlockSpec((P, 1, 1), lambda g: (g, 0, 0)),
        ],
        out_specs=[
            pl.BlockSpec((1, 1), lambda g: (0, 0)),
            pl.BlockSpec((1, 1), lambda g: (0, 0)),
        ],
        out_shape=[
            jax.ShapeDtypeStruct((1, 1), jnp.float32),
            jax.ShapeDtypeStruct((1, 1), jnp.float32),
        ],
    )(predT, pred, tag, tagT, w)

    return (loss[0, 0], lossc[0, 0])
